# baseline (speedup 1.0000x reference)
import functools

import numpy as np

import jax
import jax.numpy as jnp
from jax import lax
from jax.experimental import pallas as pl
from jax.experimental.pallas import tpu as pltpu
from jax.experimental.pallas import tpu_sc as plsc

N_USER = 30000
N_ITEM = 20000
N = N_USER + N_ITEM
D = 64
HALF = D // 2
L = 3
E = 800000
B = 4096

NC = 2
NS = 16
LANES = 16

CHUNK = 128
CH_PER_SUB = 408
E_PAD = NS * CH_PER_SUB * CHUNK
N_CHUNKS = E_PAD // CHUNK

ROWS_PER_SUB = N // NS
ZROWS = 125

_MESH = plsc.VectorSubcoreMesh(core_axis_name="c", subcore_axis_name="s")
_SC_PARAMS = pltpu.CompilerParams(use_tc_tiling_on_sc=False,
                                  needs_layout_passes=False)


def _splat(vv, e):
    return jnp.broadcast_to(vv[e], (LANES,))

NB = 2
NBATCH = CH_PER_SUB // NB
GB = 3
GI = 4


def _spmm_sc(ego_halves, cols2d, rows2d, vals2d):
    @functools.partial(
        pl.kernel,
        out_type=jax.ShapeDtypeStruct((NC, N, HALF), jnp.float32),
        mesh=_MESH,
        scratch_types=(
            [pltpu.VMEM((CHUNK,), jnp.int32)] * (GI * NB)
            + [pltpu.VMEM((CHUNK,), jnp.int32)] * (GI * NB)
            + [pltpu.VMEM((CHUNK,), jnp.float32)] * (GI * NB)
            + [pltpu.VMEM((CHUNK, HALF), jnp.float32)] * (GB * NB)
            + [
                pltpu.VMEM_SHARED((N, HALF), jnp.float32),
                pltpu.SemaphoreType.DMA,
                pltpu.SemaphoreType.DMA,
                pltpu.SemaphoreType.DMA,
                pltpu.SemaphoreType.DMA,
                pltpu.SemaphoreType.DMA,
                pltpu.SemaphoreType.DMA,
            ]
        ),
        compiler_params=_SC_PARAMS,
    )
    def k(ego_hbm, cols_hbm, rows_hbm, vals_hbm, out_hbm, *rest):
        ni = GI * NB
        cidx = [[rest[g * NB + j] for j in range(NB)] for g in range(GI)]
        ridx = [[rest[ni + g * NB + j] for j in range(NB)] for g in range(GI)]
        vals = [[rest[2 * ni + g * NB + j] for j in range(NB)]
                for g in range(GI)]
        rowb = [[rest[3 * ni + g * NB + j] for j in range(NB)]
                for g in range(GB)]
        (acc_sh, sem_i0, sem_i1, sem_g, sem_s0, sem_s1,
         sem_s2) = rest[3 * ni + GB * NB:]
        c = lax.axis_index("c")
        s = lax.axis_index("s")
        tab = ego_hbm.at[c]

        d_rows = out_hbm.at[c].at[pl.ds(0, CHUNK)]
        d_idx = cols_hbm.at[0]
        d_vals = vals_hbm.at[0]

        def drain_gathers(rg, ig, sem):
            for j in range(NB):
                pltpu.make_async_copy(tab.at[cidx[ig][j]], rowb[rg][j],
                                      sem).wait()

        def drain_scatters(rg, ig, sem):
            for j in range(NB):
                pltpu.make_async_copy(rowb[rg][j], acc_sh.at[ridx[ig][j]],
                                      sem).wait()

        def fire_idx(chb, ig, sem):
            for j in range(NB):
                pltpu.async_copy(cols_hbm.at[chb + j], cidx[ig][j], sem)
                pltpu.async_copy(rows_hbm.at[chb + j], ridx[ig][j], sem)
                pltpu.async_copy(vals_hbm.at[chb + j], vals[ig][j], sem)

        def wait_idx(ig, sem):
            for j in range(NB):
                pltpu.make_async_copy(d_idx, cidx[ig][j], sem).wait()
                pltpu.make_async_copy(d_idx, ridx[ig][j], sem).wait()
                pltpu.make_async_copy(d_vals, vals[ig][j], sem).wait()

        z16 = jnp.zeros((LANES,), jnp.float32)
        zero_v = rowb[0][0]

        @pl.loop(0, CHUNK)
        def _(r):
            zero_v[r, pl.ds(0, LANES)] = z16
            zero_v[r, pl.ds(LANES, LANES)] = z16

        @pl.loop(0, ROWS_PER_SUB // ZROWS)
        def _(i):
            pltpu.sync_copy(
                zero_v.at[pl.ds(0, ZROWS)],
                acc_sh.at[pl.ds(s * ROWS_PER_SUB + i * ZROWS, ZROWS)])

        plsc.subcore_barrier()

        ch0 = s * CH_PER_SUB
        sems_s = [sem_s0, sem_s1, sem_s2]
        sems_i = [sem_i0, sem_i1]

        fire_idx(ch0, 0, sems_i[0])
        wait_idx(0, sems_i[0])
        for j in range(NB):
            pltpu.async_copy(tab.at[cidx[0][j]], rowb[0][j], sem_g)
        fire_idx(ch0 + NB, 1, sems_i[1])

        def do_batch(b, g, gi, p):
            g1 = (g + 1) % GB
            gi1 = (gi + 1) % GI
            gi2 = (gi + 2) % GI

            @pl.when(b >= 2)
            def _():
                drain_scatters(g1, gi2, sems_s[g1])

            @pl.when(b + 2 < NBATCH)
            def _():
                fire_idx(ch0 + (b + 2) * NB, gi2, sems_i[p])

            drain_gathers(g, gi, sem_g)

            @pl.when(b + 1 < NBATCH)
            def _():
                wait_idx(gi1, sems_i[1 - p])
                for j in range(NB):
                    pltpu.async_copy(tab.at[cidx[gi1][j]], rowb[g1][j], sem_g)

            for jj in range(NB):
                vref = vals[gi][jj]
                rref = rowb[g][jj]

                @pl.loop(0, CHUNK // LANES)
                def _(gg):
                    base = gg * LANES
                    vv = vref[pl.ds(base, LANES)]
                    for e in range(LANES):
                        sp = _splat(vv, e)
                        lo = rref[base + e, pl.ds(0, LANES)]
                        hi = rref[base + e, pl.ds(LANES, LANES)]
                        rref[base + e, pl.ds(0, LANES)] = lo * sp
                        rref[base + e, pl.ds(LANES, LANES)] = hi * sp

            for j in range(NB):
                pltpu.async_copy(rowb[g][j], acc_sh.at[ridx[gi][j]],
                                 sems_s[g], add=True)

        PERIOD = 12

        @pl.loop(0, NBATCH // PERIOD)
        def _(t):
            for k in range(PERIOD):
                do_batch(PERIOD * t + k, k % GB, k % GI, k % 2)

        drain_scatters((NBATCH - 2) % GB, (NBATCH - 2) % GI,
                       sems_s[(NBATCH - 2) % GB])
        drain_scatters((NBATCH - 1) % GB, (NBATCH - 1) % GI,
                       sems_s[(NBATCH - 1) % GB])

        plsc.subcore_barrier()

        rb = s * ROWS_PER_SUB
        pltpu.sync_copy(acc_sh.at[pl.ds(rb, ROWS_PER_SUB)],
                        out_hbm.at[c].at[pl.ds(rb, ROWS_PER_SUB)])

    return k(ego_halves, cols2d, rows2d, vals2d)


BR = 2000


def _dense_tc(side, ego, W1l, b1l, W2l, b2l):
    def body(s_ref, e_ref, w1_ref, b1_ref, w2_ref, b2_ref, o_ref):
        s0 = s_ref[0]
        s1 = s_ref[1]
        e0 = e_ref[0]
        e1 = e_ref[1]
        w1 = w1_ref[...]
        w2 = w2_ref[...]

        def mm(a, w):
            return lax.dot_general(a, w, (((1,), (0,)), ((), ())),
                                   preferred_element_type=jnp.float32,
                                   precision=lax.Precision.HIGHEST)

        a = mm(s0 + e0, w1[:HALF, :]) + mm(s1 + e1, w1[HALF:, :]) + b1_ref[...]
        a = jnp.where(a >= 0, a, 0.2 * a)
        bb = mm(s0 * e0, w2[:HALF, :]) + mm(s1 * e1, w2[HALF:, :]) + b2_ref[...]
        bb = jnp.where(bb >= 0, bb, 0.2 * bb)
        t = a + bb
        nrm = jnp.sqrt(jnp.sum(t * t, axis=1, keepdims=True)) + 1e-8
        t = t / nrm
        o_ref[0] = t[:, :HALF]
        o_ref[1] = t[:, HALF:]

    return pl.pallas_call(
        body,
        grid=(N // BR,),
        in_specs=[
            pl.BlockSpec((NC, BR, HALF), lambda i: (0, i, 0)),
            pl.BlockSpec((NC, BR, HALF), lambda i: (0, i, 0)),
            pl.BlockSpec((D, D), lambda i: (0, 0)),
            pl.BlockSpec((1, D), lambda i: (0, 0)),
            pl.BlockSpec((D, D), lambda i: (0, 0)),
            pl.BlockSpec((1, D), lambda i: (0, 0)),
        ],
        out_specs=pl.BlockSpec((NC, BR, HALF), lambda i: (0, i, 0)),
        out_shape=jax.ShapeDtypeStruct((NC, N, HALF), jnp.float32),
    )(side, ego, W1l, b1l, W2l, b2l)


NIDX = 3 * B
IDX_CHUNKS = NIDX // CHUNK
CH_PER_WID = IDX_CHUNKS // (NC * NS)


def _final_gather_sc(stages, idx2d):
    @functools.partial(
        pl.kernel,
        out_type=[jax.ShapeDtypeStruct((NIDX, HALF), jnp.float32)
                  for _ in range(2 * len(stages))],
        mesh=_MESH,
        scratch_types=[
            pltpu.VMEM((CHUNK,), jnp.int32),
            pltpu.VMEM((CHUNK, HALF), jnp.float32),
            pltpu.SemaphoreType.DMA,
        ],
        compiler_params=_SC_PARAMS,
    )
    def k(s0, s1, s2, s3, i_hbm, o0, o1, o2, o3, o4, o5, o6, o7,
          idx_v, rows_v, sem):
        c = lax.axis_index("c")
        s = lax.axis_index("s")
        wid = c * NS + s
        tables = [s0, s1, s2, s3]
        outs = [o0, o1, o2, o3, o4, o5, o6, o7]

        @pl.loop(0, CH_PER_WID)
        def _(t):
            ch = wid * CH_PER_WID + t
            pltpu.sync_copy(i_hbm.at[ch], idx_v)
            for kk in range(8):
                tab = tables[kk // 2].at[kk % 2]
                pltpu.async_copy(tab.at[idx_v], rows_v, sem).wait()
                pltpu.sync_copy(rows_v, outs[kk].at[pl.ds(ch * CHUNK, CHUNK)])

    return k(*stages, idx2d)


def kernel(user_emb, item_emb, W1, b1, W2, b2, lap_vals, lap_rows, lap_cols,
           users, pos_items, neg_items):
    ego64 = jnp.concatenate([user_emb, item_emb], axis=0)
    ego = jnp.stack([ego64[:, :HALF], ego64[:, HALF:]])

    pad = E_PAD - E
    cols2d = jnp.concatenate(
        [lap_cols.astype(jnp.int32), jnp.zeros((pad,), jnp.int32)]
    ).reshape(N_CHUNKS, CHUNK)
    rows2d = jnp.concatenate(
        [lap_rows.astype(jnp.int32), jnp.zeros((pad,), jnp.int32)]
    ).reshape(N_CHUNKS, CHUNK)
    vals2d = jnp.concatenate(
        [lap_vals, jnp.zeros((pad,), jnp.float32)]
    ).reshape(N_CHUNKS, CHUNK)

    b1r = b1.reshape(L, 1, D)
    b2r = b2.reshape(L, 1, D)

    stages = [ego]
    for l in range(L):
        side = _spmm_sc(ego, cols2d, rows2d, vals2d)
        ego = _dense_tc(side, ego, W1[l], b1r[l], W2[l], b2r[l])
        stages.append(ego)

    idx_all = jnp.concatenate([
        users.astype(jnp.int32),
        pos_items.astype(jnp.int32) + N_USER,
        neg_items.astype(jnp.int32) + N_USER,
    ]).reshape(IDX_CHUNKS, CHUNK)

    outs8 = _final_gather_sc(stages, idx_all)
    all_g = jnp.concatenate(outs8, axis=1)
    return all_g.reshape(3, B, (L + 1) * D)


# speedup vs baseline: 1.2690x; 1.2690x over previous
import functools

import numpy as np

import jax
import jax.numpy as jnp
from jax import lax
from jax.experimental import pallas as pl
from jax.experimental.pallas import tpu as pltpu
from jax.experimental.pallas import tpu_sc as plsc

N_USER = 30000
N_ITEM = 20000
N = N_USER + N_ITEM
D = 64
HALF = D // 2
L = 3
E = 800000
B = 4096

NC = 2
NS = 16
LANES = 16

CHUNK = 128
CH_PER_SUB = 402
E_PAD = NS * CH_PER_SUB * CHUNK
N_CHUNKS = E_PAD // CHUNK

ROWS_PER_SUB = N // NS
ZROWS = 125

_MESH = plsc.VectorSubcoreMesh(core_axis_name="c", subcore_axis_name="s")
_SC_PARAMS = pltpu.CompilerParams(use_tc_tiling_on_sc=False,
                                  needs_layout_passes=False)


def _splat(vv, e):
    return jnp.broadcast_to(vv[e], (LANES,))

NB = 2
NBATCH = CH_PER_SUB // NB
GB = 3
GI = 3


def _spmm_sc(ego_halves, cols2d, rows2d, vals2d):
    @functools.partial(
        pl.kernel,
        out_type=jax.ShapeDtypeStruct((NC, N, HALF), jnp.float32),
        mesh=_MESH,
        scratch_types=(
            [pltpu.VMEM((CHUNK,), jnp.int32)] * (GI * NB)
            + [pltpu.VMEM((CHUNK,), jnp.int32)] * (GI * NB)
            + [pltpu.VMEM((CHUNK,), jnp.float32)] * (GI * NB)
            + [pltpu.VMEM((CHUNK, HALF), jnp.float32)] * (GB * NB)
            + [
                pltpu.VMEM_SHARED((N, HALF), jnp.float32),
                pltpu.SemaphoreType.DMA,
                pltpu.SemaphoreType.DMA,
                pltpu.SemaphoreType.DMA,
                pltpu.SemaphoreType.DMA,
                pltpu.SemaphoreType.DMA,
                pltpu.SemaphoreType.DMA,
            ]
        ),
        compiler_params=_SC_PARAMS,
    )
    def k(ego_hbm, cols_hbm, rows_hbm, vals_hbm, out_hbm, *rest):
        ni = GI * NB
        cidx = [[rest[g * NB + j] for j in range(NB)] for g in range(GI)]
        ridx = [[rest[ni + g * NB + j] for j in range(NB)] for g in range(GI)]
        vals = [[rest[2 * ni + g * NB + j] for j in range(NB)]
                for g in range(GI)]
        rowb = [[rest[3 * ni + g * NB + j] for j in range(NB)]
                for g in range(GB)]
        (acc_sh, sem_i0, sem_i1, sem_g, sem_s0, sem_s1,
         sem_s2) = rest[3 * ni + GB * NB:]
        c = lax.axis_index("c")
        s = lax.axis_index("s")
        tab = ego_hbm.at[c]

        d_rows = out_hbm.at[c].at[pl.ds(0, CHUNK)]
        d_idx = cols_hbm.at[0]
        d_vals = vals_hbm.at[0]

        def drain_gathers(rg, ig, sem):
            for j in range(NB):
                pltpu.make_async_copy(tab.at[cidx[ig][j]], rowb[rg][j],
                                      sem).wait()

        def drain_scatters(rg, ig, sem):
            for j in range(NB):
                pltpu.make_async_copy(rowb[rg][j], acc_sh.at[ridx[ig][j]],
                                      sem).wait()

        def fire_idx(chb, ig, sem):
            for j in range(NB):
                pltpu.async_copy(cols_hbm.at[chb + j], cidx[ig][j], sem)
                pltpu.async_copy(rows_hbm.at[chb + j], ridx[ig][j], sem)
                pltpu.async_copy(vals_hbm.at[chb + j], vals[ig][j], sem)

        def wait_idx(ig, sem):
            for j in range(NB):
                pltpu.make_async_copy(d_idx, cidx[ig][j], sem).wait()
                pltpu.make_async_copy(d_idx, ridx[ig][j], sem).wait()
                pltpu.make_async_copy(d_vals, vals[ig][j], sem).wait()

        z16 = jnp.zeros((LANES,), jnp.float32)
        zero_v = rowb[0][0]

        @pl.loop(0, CHUNK)
        def _(r):
            zero_v[r, pl.ds(0, LANES)] = z16
            zero_v[r, pl.ds(LANES, LANES)] = z16

        @pl.loop(0, ROWS_PER_SUB // ZROWS)
        def _(i):
            pltpu.sync_copy(
                zero_v.at[pl.ds(0, ZROWS)],
                acc_sh.at[pl.ds(s * ROWS_PER_SUB + i * ZROWS, ZROWS)])

        plsc.subcore_barrier()

        ch0 = s * CH_PER_SUB
        sems_s = [sem_s0, sem_s1, sem_s2]
        sems_i = [sem_i0, sem_i1]

        fire_idx(ch0, 0, sems_i[0])
        wait_idx(0, sems_i[0])
        for j in range(NB):
            pltpu.async_copy(tab.at[cidx[0][j]], rowb[0][j], sem_g)

        def do_batch(b, g):
            gn = (g + 1) % GB

            @pl.when(b >= 2)
            def _():
                drain_scatters(gn, gn, sems_s[gn])

            @pl.when(b + 1 < NBATCH)
            def _():
                fire_idx(ch0 + (b + 1) * NB, gn, sems_i[0])

            drain_gathers(g, g, sem_g)

            @pl.when(b + 1 < NBATCH)
            def _():
                wait_idx(gn, sems_i[0])
                for j in range(NB):
                    pltpu.async_copy(tab.at[cidx[gn][j]], rowb[gn][j], sem_g)

            for jj in range(NB):
                vref = vals[g][jj]
                rref = rowb[g][jj]

                @pl.loop(0, CHUNK // LANES)
                def _(gg):
                    base = gg * LANES
                    vv = vref[pl.ds(base, LANES)]
                    for e in range(LANES):
                        sp = _splat(vv, e)
                        lo = rref[base + e, pl.ds(0, LANES)]
                        hi = rref[base + e, pl.ds(LANES, LANES)]
                        rref[base + e, pl.ds(0, LANES)] = lo * sp
                        rref[base + e, pl.ds(LANES, LANES)] = hi * sp

            for j in range(NB):
                pltpu.async_copy(rowb[g][j], acc_sh.at[ridx[g][j]],
                                 sems_s[g], add=True)

        @pl.loop(0, NBATCH // GB)
        def _(t):
            do_batch(GB * t, 0)
            do_batch(GB * t + 1, 1)
            do_batch(GB * t + 2, 2)

        drain_scatters((NBATCH - 2) % GB, (NBATCH - 2) % GB,
                       sems_s[(NBATCH - 2) % GB])
        drain_scatters((NBATCH - 1) % GB, (NBATCH - 1) % GB,
                       sems_s[(NBATCH - 1) % GB])

        plsc.subcore_barrier()

        rb = s * ROWS_PER_SUB
        pltpu.sync_copy(acc_sh.at[pl.ds(rb, ROWS_PER_SUB)],
                        out_hbm.at[c].at[pl.ds(rb, ROWS_PER_SUB)])

    return k(ego_halves, cols2d, rows2d, vals2d)


BR = 2000


def _dense_tc(side, ego, W1l, b1l, W2l, b2l):
    def body(s_ref, e_ref, w1_ref, b1_ref, w2_ref, b2_ref, o_ref):
        s0 = s_ref[0]
        s1 = s_ref[1]
        e0 = e_ref[0]
        e1 = e_ref[1]
        w1 = w1_ref[...]
        w2 = w2_ref[...]

        def mm(a, w):
            return lax.dot_general(a, w, (((1,), (0,)), ((), ())),
                                   preferred_element_type=jnp.float32,
                                   precision=lax.Precision.HIGHEST)

        a = mm(s0 + e0, w1[:HALF, :]) + mm(s1 + e1, w1[HALF:, :]) + b1_ref[...]
        a = jnp.where(a >= 0, a, 0.2 * a)
        bb = mm(s0 * e0, w2[:HALF, :]) + mm(s1 * e1, w2[HALF:, :]) + b2_ref[...]
        bb = jnp.where(bb >= 0, bb, 0.2 * bb)
        t = a + bb
        nrm = jnp.sqrt(jnp.sum(t * t, axis=1, keepdims=True)) + 1e-8
        t = t / nrm
        o_ref[0] = t[:, :HALF]
        o_ref[1] = t[:, HALF:]

    return pl.pallas_call(
        body,
        grid=(N // BR,),
        in_specs=[
            pl.BlockSpec((NC, BR, HALF), lambda i: (0, i, 0)),
            pl.BlockSpec((NC, BR, HALF), lambda i: (0, i, 0)),
            pl.BlockSpec((D, D), lambda i: (0, 0)),
            pl.BlockSpec((1, D), lambda i: (0, 0)),
            pl.BlockSpec((D, D), lambda i: (0, 0)),
            pl.BlockSpec((1, D), lambda i: (0, 0)),
        ],
        out_specs=pl.BlockSpec((NC, BR, HALF), lambda i: (0, i, 0)),
        out_shape=jax.ShapeDtypeStruct((NC, N, HALF), jnp.float32),
    )(side, ego, W1l, b1l, W2l, b2l)


NIDX = 3 * B
IDX_CHUNKS = NIDX // CHUNK
CH_PER_WID = IDX_CHUNKS // (NC * NS)


def _final_gather_sc(stages, idx2d):
    @functools.partial(
        pl.kernel,
        out_type=[jax.ShapeDtypeStruct((NIDX, HALF), jnp.float32)
                  for _ in range(2 * len(stages))],
        mesh=_MESH,
        scratch_types=[
            pltpu.VMEM((CHUNK,), jnp.int32),
            pltpu.VMEM((CHUNK, HALF), jnp.float32),
            pltpu.SemaphoreType.DMA,
        ],
        compiler_params=_SC_PARAMS,
    )
    def k(s0, s1, s2, s3, i_hbm, o0, o1, o2, o3, o4, o5, o6, o7,
          idx_v, rows_v, sem):
        c = lax.axis_index("c")
        s = lax.axis_index("s")
        wid = c * NS + s
        tables = [s0, s1, s2, s3]
        outs = [o0, o1, o2, o3, o4, o5, o6, o7]

        @pl.loop(0, CH_PER_WID)
        def _(t):
            ch = wid * CH_PER_WID + t
            pltpu.sync_copy(i_hbm.at[ch], idx_v)
            for kk in range(8):
                tab = tables[kk // 2].at[kk % 2]
                pltpu.async_copy(tab.at[idx_v], rows_v, sem).wait()
                pltpu.sync_copy(rows_v, outs[kk].at[pl.ds(ch * CHUNK, CHUNK)])

    return k(*stages, idx2d)


def kernel(user_emb, item_emb, W1, b1, W2, b2, lap_vals, lap_rows, lap_cols,
           users, pos_items, neg_items):
    ego64 = jnp.concatenate([user_emb, item_emb], axis=0)
    ego = jnp.stack([ego64[:, :HALF], ego64[:, HALF:]])

    pad = E_PAD - E
    cols2d = jnp.concatenate(
        [lap_cols.astype(jnp.int32), jnp.zeros((pad,), jnp.int32)]
    ).reshape(N_CHUNKS, CHUNK)
    rows2d = jnp.concatenate(
        [lap_rows.astype(jnp.int32), jnp.zeros((pad,), jnp.int32)]
    ).reshape(N_CHUNKS, CHUNK)
    vals2d = jnp.concatenate(
        [lap_vals, jnp.zeros((pad,), jnp.float32)]
    ).reshape(N_CHUNKS, CHUNK)

    b1r = b1.reshape(L, 1, D)
    b2r = b2.reshape(L, 1, D)

    stages = [ego]
    for l in range(L):
        side = _spmm_sc(ego, cols2d, rows2d, vals2d)
        ego = _dense_tc(side, ego, W1[l], b1r[l], W2[l], b2r[l])
        stages.append(ego)

    idx_all = jnp.concatenate([
        users.astype(jnp.int32),
        pos_items.astype(jnp.int32) + N_USER,
        neg_items.astype(jnp.int32) + N_USER,
    ]).reshape(IDX_CHUNKS, CHUNK)

    outs8 = _final_gather_sc(stages, idx_all)
    all_g = jnp.concatenate(outs8, axis=1)
    return all_g.reshape(3, B, (L + 1) * D)


# speedup vs baseline: 1.2693x; 1.0002x over previous
import functools

import jax
import jax.numpy as jnp
from jax import lax
from jax.experimental import pallas as pl
from jax.experimental.pallas import tpu as pltpu
from jax.experimental.pallas import tpu_sc as plsc

N_USER = 30000
N_ITEM = 20000
N = N_USER + N_ITEM
D = 64
HALF = D // 2
L = 3
E = 800000
B = 4096

NC = 2
NS = 16
LANES = 16

CHUNK = 128
CH_PER_SUB = 402
E_PAD = NS * CH_PER_SUB * CHUNK
N_CHUNKS = E_PAD // CHUNK

ROWS_PER_SUB = N // NS
ZROWS = 125

_MESH = plsc.VectorSubcoreMesh(core_axis_name="c", subcore_axis_name="s")
_SC_PARAMS = pltpu.CompilerParams(use_tc_tiling_on_sc=False,
                                  needs_layout_passes=False)


def _splat(vv, e):
    return jnp.broadcast_to(vv[e], (LANES,))

NB = 2
NBATCH = CH_PER_SUB // NB
GB = 3
GI = 3


def _spmm_sc(ego_halves, cols2d, rows2d, vals2d):
    @functools.partial(
        pl.kernel,
        out_type=jax.ShapeDtypeStruct((NC, N, HALF), jnp.float32),
        mesh=_MESH,
        scratch_types=(
            [pltpu.VMEM((CHUNK,), jnp.int32)] * (GI * NB)
            + [pltpu.VMEM((CHUNK,), jnp.int32)] * (GI * NB)
            + [pltpu.VMEM((CHUNK,), jnp.float32)] * (GI * NB)
            + [pltpu.VMEM((CHUNK, HALF), jnp.float32)] * (GB * NB)
            + [
                pltpu.VMEM_SHARED((N, HALF), jnp.float32),
                pltpu.SemaphoreType.DMA,
                pltpu.SemaphoreType.DMA,
                pltpu.SemaphoreType.DMA,
                pltpu.SemaphoreType.DMA,
                pltpu.SemaphoreType.DMA,
                pltpu.SemaphoreType.DMA,
            ]
        ),
        compiler_params=_SC_PARAMS,
    )
    def k(ego_hbm, cols_hbm, rows_hbm, vals_hbm, out_hbm, *rest):
        ni = GI * NB
        cidx = [[rest[g * NB + j] for j in range(NB)] for g in range(GI)]
        ridx = [[rest[ni + g * NB + j] for j in range(NB)] for g in range(GI)]
        vals = [[rest[2 * ni + g * NB + j] for j in range(NB)]
                for g in range(GI)]
        rowb = [[rest[3 * ni + g * NB + j] for j in range(NB)]
                for g in range(GB)]
        (acc_sh, sem_i0, sem_i1, sem_g, sem_s0, sem_s1,
         sem_s2) = rest[3 * ni + GB * NB:]
        c = lax.axis_index("c")
        s = lax.axis_index("s")
        tab = ego_hbm.at[c]

        d_idx = cols_hbm.at[0]
        d_vals = vals_hbm.at[0]

        def drain_gathers(rg, ig, sem):
            for j in range(NB):
                pltpu.make_async_copy(tab.at[cidx[ig][j]], rowb[rg][j],
                                      sem).wait()

        def drain_scatters(rg, ig, sem):
            for j in range(NB):
                pltpu.make_async_copy(rowb[rg][j], acc_sh.at[ridx[ig][j]],
                                      sem).wait()

        def fire_idx(chb, ig, sem):
            for j in range(NB):
                pltpu.async_copy(cols_hbm.at[chb + j], cidx[ig][j], sem)
                pltpu.async_copy(rows_hbm.at[chb + j], ridx[ig][j], sem)
                pltpu.async_copy(vals_hbm.at[chb + j], vals[ig][j], sem)

        def wait_idx(ig, sem):
            for j in range(NB):
                pltpu.make_async_copy(d_idx, cidx[ig][j], sem).wait()
                pltpu.make_async_copy(d_idx, ridx[ig][j], sem).wait()
                pltpu.make_async_copy(d_vals, vals[ig][j], sem).wait()

        z16 = jnp.zeros((LANES,), jnp.float32)
        zero_v = rowb[0][0]

        @pl.loop(0, CHUNK)
        def _(r):
            zero_v[r, pl.ds(0, LANES)] = z16
            zero_v[r, pl.ds(LANES, LANES)] = z16

        @pl.loop(0, ROWS_PER_SUB // ZROWS)
        def _(i):
            pltpu.sync_copy(
                zero_v.at[pl.ds(0, ZROWS)],
                acc_sh.at[pl.ds(s * ROWS_PER_SUB + i * ZROWS, ZROWS)])

        plsc.subcore_barrier()

        ch0 = s * CH_PER_SUB
        sems_s = [sem_s0, sem_s1, sem_s2]
        sems_i = [sem_i0, sem_i1]

        fire_idx(ch0, 0, sems_i[0])
        wait_idx(0, sems_i[0])
        for j in range(NB):
            pltpu.async_copy(tab.at[cidx[0][j]], rowb[0][j], sem_g)

        def do_batch(b, g):
            gn = (g + 1) % GB

            @pl.when(b >= 2)
            def _():
                drain_scatters(gn, gn, sems_s[gn])

            @pl.when(b + 1 < NBATCH)
            def _():
                fire_idx(ch0 + (b + 1) * NB, gn, sems_i[0])

            drain_gathers(g, g, sem_g)

            @pl.when(b + 1 < NBATCH)
            def _():
                wait_idx(gn, sems_i[0])
                for j in range(NB):
                    pltpu.async_copy(tab.at[cidx[gn][j]], rowb[gn][j], sem_g)

            for jj in range(NB):
                vref = vals[g][jj]
                rref = rowb[g][jj]

                @pl.loop(0, CHUNK // LANES)
                def _(gg):
                    base = gg * LANES
                    vv = vref[pl.ds(base, LANES)]
                    for e in range(LANES):
                        sp = _splat(vv, e)
                        lo = rref[base + e, pl.ds(0, LANES)]
                        hi = rref[base + e, pl.ds(LANES, LANES)]
                        rref[base + e, pl.ds(0, LANES)] = lo * sp
                        rref[base + e, pl.ds(LANES, LANES)] = hi * sp

            for j in range(NB):
                pltpu.async_copy(rowb[g][j], acc_sh.at[ridx[g][j]],
                                 sems_s[g], add=True)

        @pl.loop(0, NBATCH // GB)
        def _(t):
            do_batch(GB * t, 0)
            do_batch(GB * t + 1, 1)
            do_batch(GB * t + 2, 2)

        drain_scatters((NBATCH - 2) % GB, (NBATCH - 2) % GB,
                       sems_s[(NBATCH - 2) % GB])
        drain_scatters((NBATCH - 1) % GB, (NBATCH - 1) % GB,
                       sems_s[(NBATCH - 1) % GB])

        plsc.subcore_barrier()

        rb = s * ROWS_PER_SUB
        pltpu.sync_copy(acc_sh.at[pl.ds(rb, ROWS_PER_SUB)],
                        out_hbm.at[c].at[pl.ds(rb, ROWS_PER_SUB)])

    return k(ego_halves, cols2d, rows2d, vals2d)


BR = 2000


def _dense_tc(side, ego, W1l, b1l, W2l, b2l):
    def body(s_ref, e_ref, w1_ref, b1_ref, w2_ref, b2_ref, o_ref):
        s0 = s_ref[0]
        s1 = s_ref[1]
        e0 = e_ref[0]
        e1 = e_ref[1]
        w1 = w1_ref[...]
        w2 = w2_ref[...]

        def mm(a, w):
            return lax.dot_general(a, w, (((1,), (0,)), ((), ())),
                                   preferred_element_type=jnp.float32,
                                   precision=lax.Precision.HIGHEST)

        a = mm(s0 + e0, w1[:HALF, :]) + mm(s1 + e1, w1[HALF:, :]) + b1_ref[...]
        a = jnp.where(a >= 0, a, 0.2 * a)
        bb = mm(s0 * e0, w2[:HALF, :]) + mm(s1 * e1, w2[HALF:, :]) + b2_ref[...]
        bb = jnp.where(bb >= 0, bb, 0.2 * bb)
        t = a + bb
        nrm = jnp.sqrt(jnp.sum(t * t, axis=1, keepdims=True)) + 1e-8
        t = t / nrm
        o_ref[0] = t[:, :HALF]
        o_ref[1] = t[:, HALF:]

    return pl.pallas_call(
        body,
        grid=(N // BR,),
        in_specs=[
            pl.BlockSpec((NC, BR, HALF), lambda i: (0, i, 0)),
            pl.BlockSpec((NC, BR, HALF), lambda i: (0, i, 0)),
            pl.BlockSpec((D, D), lambda i: (0, 0)),
            pl.BlockSpec((1, D), lambda i: (0, 0)),
            pl.BlockSpec((D, D), lambda i: (0, 0)),
            pl.BlockSpec((1, D), lambda i: (0, 0)),
        ],
        out_specs=pl.BlockSpec((NC, BR, HALF), lambda i: (0, i, 0)),
        out_shape=jax.ShapeDtypeStruct((NC, N, HALF), jnp.float32),
    )(side, ego, W1l, b1l, W2l, b2l)


NIDX = 3 * B
IDX_CHUNKS = NIDX // CHUNK
CH_PER_WID = IDX_CHUNKS // (NC * NS)


def _final_gather_sc(stages, idx2d):
    @functools.partial(
        pl.kernel,
        out_type=[jax.ShapeDtypeStruct((NIDX, HALF), jnp.float32)
                  for _ in range(2 * len(stages))],
        mesh=_MESH,
        scratch_types=[
            pltpu.VMEM((CHUNK,), jnp.int32),
            pltpu.VMEM((CHUNK, HALF), jnp.float32),
            pltpu.SemaphoreType.DMA,
        ],
        compiler_params=_SC_PARAMS,
    )
    def k(s0, s1, s2, s3, i_hbm, o0, o1, o2, o3, o4, o5, o6, o7,
          idx_v, rows_v, sem):
        c = lax.axis_index("c")
        s = lax.axis_index("s")
        wid = c * NS + s
        tables = [s0, s1, s2, s3]
        outs = [o0, o1, o2, o3, o4, o5, o6, o7]

        @pl.loop(0, CH_PER_WID)
        def _(t):
            ch = wid * CH_PER_WID + t
            pltpu.sync_copy(i_hbm.at[ch], idx_v)
            for kk in range(8):
                tab = tables[kk // 2].at[kk % 2]
                pltpu.async_copy(tab.at[idx_v], rows_v, sem).wait()
                pltpu.sync_copy(rows_v, outs[kk].at[pl.ds(ch * CHUNK, CHUNK)])

    return k(*stages, idx2d)


def kernel(user_emb, item_emb, W1, b1, W2, b2, lap_vals, lap_rows, lap_cols,
           users, pos_items, neg_items):
    ego64 = jnp.concatenate([user_emb, item_emb], axis=0)
    ego = jnp.stack([ego64[:, :HALF], ego64[:, HALF:]])

    pad = E_PAD - E
    cols2d = jnp.concatenate(
        [lap_cols.astype(jnp.int32), jnp.zeros((pad,), jnp.int32)]
    ).reshape(N_CHUNKS, CHUNK)
    rows2d = jnp.concatenate(
        [lap_rows.astype(jnp.int32), jnp.zeros((pad,), jnp.int32)]
    ).reshape(N_CHUNKS, CHUNK)
    vals2d = jnp.concatenate(
        [lap_vals, jnp.zeros((pad,), jnp.float32)]
    ).reshape(N_CHUNKS, CHUNK)

    b1r = b1.reshape(L, 1, D)
    b2r = b2.reshape(L, 1, D)

    stages = [ego]
    for l in range(L):
        side = _spmm_sc(ego, cols2d, rows2d, vals2d)
        ego = _dense_tc(side, ego, W1[l], b1r[l], W2[l], b2r[l])
        stages.append(ego)

    idx_all = jnp.concatenate([
        users.astype(jnp.int32),
        pos_items.astype(jnp.int32) + N_USER,
        neg_items.astype(jnp.int32) + N_USER,
    ]).reshape(IDX_CHUNKS, CHUNK)

    outs8 = _final_gather_sc(stages, idx_all)
    all_g = jnp.concatenate(outs8, axis=1)
    return all_g.reshape(3, B, (L + 1) * D)
